# baseline (speedup 1.0000x reference)
import functools

import jax
import jax.numpy as jnp
from jax import lax
from jax.experimental import pallas as pl
from jax.experimental.pallas import tpu as pltpu
from jax.experimental.pallas import tpu_sc as plsc

_NUM_CLASSES = 1000
_FEAT = 128
_BATCH = 4096
_M = 0.99
_TEMP = 0.1
_BASE_TEMP = 0.1

_NW = 32
_CPW = 32
_CPAD = _NW * _CPW
_CHW = 64
_NBUF = 8
_WAVE = _CHW * _NBUF
_NVEC = _FEAT // 16
_MIDX_SZ = _BATCH + _WAVE + 16


def _sc_ema_body(feat_hbm, lab_hbm, proto_hbm, out_hbm,
                 lab_v, midx_v, prot_v, rows_v, sem):
    wid = lax.axis_index("s") * 2 + lax.axis_index("c")
    lo = wid * _CPW
    hi = lo + _CPW

    pltpu.sync_copy(lab_hbm, lab_v.at[pl.ds(0, _BATCH)])
    pltpu.sync_copy(proto_hbm.at[pl.ds(lo, _CPW)], prot_v)

    zero16 = jnp.zeros((16,), jnp.int32)

    def zbody(i, c):
        midx_v[pl.ds(pl.multiple_of(i * 16, 16), 16)] = zero16
        return c

    lax.fori_loop(0, _MIDX_SZ // 16, zbody, 0, unroll=False)

    pow2 = jnp.left_shift(jnp.int32(1), lax.iota(jnp.int32, 16))

    def sbody(i, cnt):
        base = i * 16
        lv = lab_v[pl.ds(pl.multiple_of(base, 16), 16)]
        m = (lv >= lo) & (lv < hi)
        w = jnp.where(m, pow2, zero16)
        pr = w + lax.rev(w, (0,))
        bits = pr[0]
        for k in range(1, 8):
            bits = bits + pr[k]

        def append(c):
            for k in range(16):
                midx_v[pl.ds(c, 16)] = jnp.full((16,), base + k, jnp.int32)
                c = c + ((bits >> k) & 1)
            return c

        return lax.cond(bits != 0, append, lambda c: c, cnt)

    cnt = lax.fori_loop(0, _BATCH // 16, sbody, 0, unroll=False)

    def wave(wi, c):
        wbase = pl.multiple_of(wi * _WAVE, _WAVE)
        for b in range(_NBUF):
            pltpu.async_copy(
                feat_hbm.at[midx_v.at[pl.ds(wbase + b * _CHW, _CHW)]],
                rows_v.at[pl.ds(b * _CHW, _CHW)], sem)
        for b in range(_NBUF):
            pltpu.make_async_copy(
                feat_hbm.at[midx_v.at[pl.ds(wbase + b * _CHW, _CHW)]],
                rows_v.at[pl.ds(b * _CHW, _CHW)], sem).wait()
        jhi = jnp.minimum(cnt - wbase, _WAVE)

        def ebody(j, cc):
            idx = midx_v[pl.ds(wbase + j, 16)][0]
            lloc = lab_v[pl.ds(idx, 16)][0] - lo
            ts = []
            ss = jnp.zeros((16,), jnp.float32)
            for k in range(_NVEC):
                p = prot_v[lloc, pl.ds(k * 16, 16)]
                f = rows_v[j, pl.ds(k * 16, 16)]
                t = p * _M + f * (1.0 - _M)
                ts.append(t)
                ss = ss + t * t
            ss = ss + lax.rev(ss, (0,))
            s = ss[0]
            for k in range(1, 8):
                s = s + ss[k]
            s = jnp.maximum(s, 1e-30)
            ib = lax.bitcast_convert_type(s, jnp.int32)
            y = lax.bitcast_convert_type(
                jnp.int32(0x5F3759DF) - (ib >> 1), jnp.float32)
            for _ in range(3):
                y = y * (1.5 - 0.5 * s * y * y)
            nrm = s * y
            scale = jnp.where(nrm > 1e-12, y, 1e12)
            for k in range(_NVEC):
                prot_v[lloc, pl.ds(k * 16, 16)] = ts[k] * scale
            return cc

        lax.fori_loop(0, jhi, ebody, 0, unroll=False)
        return c

    nwave = (cnt + _WAVE - 1) // _WAVE
    lax.fori_loop(0, nwave, wave, 0, unroll=False)

    pltpu.sync_copy(prot_v, out_hbm.at[pl.ds(lo, _CPW)])


_sc_ema = functools.partial(
    pl.kernel,
    out_type=jax.ShapeDtypeStruct((_CPAD, _FEAT), jnp.float32),
    mesh=plsc.VectorSubcoreMesh(core_axis_name="c", subcore_axis_name="s"),
    scratch_types=[
        pltpu.VMEM((_BATCH + 16,), jnp.int32),
        pltpu.VMEM((_MIDX_SZ,), jnp.int32),
        pltpu.VMEM((_CPW, _FEAT), jnp.float32),
        pltpu.VMEM((_WAVE, _FEAT), jnp.float32),
        pltpu.SemaphoreType.DMA,
    ],
)(_sc_ema_body)


def _loss_body(proto_ref, out_ref):
    p = proto_ref[...]
    logits = lax.dot_general(
        p, p, (((1,), (1,)), ((), ())), preferred_element_type=jnp.float32
    ) * (1.0 / _TEMP)
    e = jnp.exp(logits)
    ii = lax.broadcasted_iota(jnp.int32, (_NUM_CLASSES, _NUM_CLASSES), 0)
    jj = lax.broadcasted_iota(jnp.int32, (_NUM_CLASSES, _NUM_CLASSES), 1)
    offdiag = ii != jj
    rowsum = jnp.sum(jnp.where(offdiag, e, 0.0), axis=1)
    diag_e = jnp.max(jnp.where(offdiag, 0.0, e), axis=1)
    mpn = jnp.log(rowsum * (1.0 / (_NUM_CLASSES - 1)))
    valid = jnp.isfinite(diag_e)
    num = jnp.sum(jnp.where(valid, mpn, 0.0))
    den = jnp.maximum(jnp.sum(valid.astype(jnp.int32)), 1).astype(jnp.float32)
    out_ref[0, 0] = (_TEMP / _BASE_TEMP) * num / den


def kernel(features, labels, prototypes):
    labels = labels.astype(jnp.int32)
    protos_pad = jnp.concatenate(
        [prototypes, jnp.zeros((_CPAD - _NUM_CLASSES, _FEAT), jnp.float32)], 0
    )
    protos = _sc_ema(features, labels, protos_pad)
    out = pl.pallas_call(
        _loss_body,
        out_shape=jax.ShapeDtypeStruct((1, 1), jnp.float32),
        in_specs=[pl.BlockSpec(memory_space=pltpu.VMEM)],
        out_specs=pl.BlockSpec(memory_space=pltpu.SMEM),
    )(protos[:_NUM_CLASSES])
    return out[0, 0]


# speedup vs baseline: 7.0517x; 7.0517x over previous
import functools

import jax
import jax.numpy as jnp
from jax import lax
from jax.experimental import pallas as pl
from jax.experimental.pallas import tpu as pltpu
from jax.experimental.pallas import tpu_sc as plsc

_NUM_CLASSES = 1000
_FEAT = 128
_BATCH = 4096
_M = 0.99
_TEMP = 0.1
_BASE_TEMP = 0.1

_NW = 32
_CPW = 32
_CPAD = _NW * _CPW
_NVEC = _FEAT // 16
_MIDX_SZ = _BATCH + 16


def _sc_ema_body(feat_hbm, lab_hbm, proto_hbm, out_hbm,
                 lab_v, midx_v, prot_v, rowb, feat_sh, sem):
    sid = lax.axis_index("s")
    wid = sid * 2 + lax.axis_index("c")
    lo = wid * _CPW
    hi = lo + _CPW

    pltpu.sync_copy(lab_hbm, lab_v.at[pl.ds(0, _BATCH)])
    pltpu.sync_copy(proto_hbm.at[pl.ds(lo, _CPW)], prot_v)

    @pl.when(sid == 0)
    def _():
        pltpu.sync_copy(feat_hbm, feat_sh)

    zero16 = jnp.zeros((16,), jnp.int32)

    pow2 = jnp.left_shift(jnp.int32(1), lax.iota(jnp.int32, 16))

    def sbody(i, cnt):
        base = i * 16
        lv = lab_v[pl.ds(pl.multiple_of(base, 16), 16)]
        m = (lv >= lo) & (lv < hi)
        w = jnp.where(m, pow2, zero16)
        pr = w + lax.rev(w, (0,))
        bits = pr[0]
        for k in range(1, 8):
            bits = bits + pr[k]

        def append(c):
            for k in range(16):
                midx_v[pl.ds(c, 16)] = jnp.full((16,), base + k, jnp.int32)
                c = c + ((bits >> k) & 1)
            return c

        return lax.cond(bits != 0, append, lambda c: c, cnt)

    cnt = lax.fori_loop(0, _BATCH // 16, sbody, 0, unroll=False)

    plsc.subcore_barrier()

    if True:
        def ebody(j, cc):
            idx = midx_v[pl.ds(j, 16)][0]
            pltpu.sync_copy(feat_sh.at[pl.ds(idx, 1)], rowb)
            lloc = lab_v[pl.ds(idx, 16)][0] - lo
            ts = []
            ss = jnp.zeros((16,), jnp.float32)
            for k in range(_NVEC):
                p = prot_v[lloc, pl.ds(k * 16, 16)]
                f = rowb[0, pl.ds(k * 16, 16)]
                t = p * _M + f * (1.0 - _M)
                ts.append(t)
                ss = ss + t * t
            ss = ss + lax.rev(ss, (0,))
            s = ss[0]
            for k in range(1, 8):
                s = s + ss[k]
            s = jnp.maximum(s, 1e-30)
            ib = lax.bitcast_convert_type(s, jnp.int32)
            y = lax.bitcast_convert_type(
                jnp.int32(0x5F3759DF) - (ib >> 1), jnp.float32)
            for _ in range(3):
                y = y * (1.5 - 0.5 * s * y * y)
            nrm = s * y
            scale = jnp.where(nrm > 1e-12, y, 1e12)
            for k in range(_NVEC):
                prot_v[lloc, pl.ds(k * 16, 16)] = ts[k] * scale
            return cc

        lax.fori_loop(0, cnt, ebody, 0, unroll=False)

    pltpu.sync_copy(prot_v, out_hbm.at[pl.ds(lo, _CPW)])


_sc_ema = functools.partial(
    pl.kernel,
    out_type=jax.ShapeDtypeStruct((_CPAD, _FEAT), jnp.float32),
    mesh=plsc.VectorSubcoreMesh(core_axis_name="c", subcore_axis_name="s"),
    scratch_types=[
        pltpu.VMEM((_BATCH + 16,), jnp.int32),
        pltpu.VMEM((_MIDX_SZ,), jnp.int32),
        pltpu.VMEM((_CPW, _FEAT), jnp.float32),
        pltpu.VMEM((1, _FEAT), jnp.float32),
        pltpu.VMEM_SHARED((_BATCH, _FEAT), jnp.float32),
        pltpu.SemaphoreType.DMA,
    ],
)(_sc_ema_body)


def _loss_body(proto_ref, out_ref):
    p = proto_ref[...]
    logits = lax.dot_general(
        p, p, (((1,), (1,)), ((), ())), preferred_element_type=jnp.float32
    ) * (1.0 / _TEMP)
    e = jnp.exp(logits)
    ii = lax.broadcasted_iota(jnp.int32, (_NUM_CLASSES, _NUM_CLASSES), 0)
    jj = lax.broadcasted_iota(jnp.int32, (_NUM_CLASSES, _NUM_CLASSES), 1)
    offdiag = ii != jj
    rowsum = jnp.sum(jnp.where(offdiag, e, 0.0), axis=1)
    diag_e = jnp.max(jnp.where(offdiag, 0.0, e), axis=1)
    mpn = jnp.log(rowsum * (1.0 / (_NUM_CLASSES - 1)))
    valid = jnp.isfinite(diag_e)
    num = jnp.sum(jnp.where(valid, mpn, 0.0))
    den = jnp.maximum(jnp.sum(valid.astype(jnp.int32)), 1).astype(jnp.float32)
    out_ref[0, 0] = (_TEMP / _BASE_TEMP) * num / den


def kernel(features, labels, prototypes):
    labels = labels.astype(jnp.int32)
    protos_pad = jnp.concatenate(
        [prototypes, jnp.zeros((_CPAD - _NUM_CLASSES, _FEAT), jnp.float32)], 0
    )
    protos = _sc_ema(features, labels, protos_pad)
    out = pl.pallas_call(
        _loss_body,
        out_shape=jax.ShapeDtypeStruct((1, 1), jnp.float32),
        in_specs=[pl.BlockSpec(memory_space=pltpu.VMEM)],
        out_specs=pl.BlockSpec(memory_space=pltpu.SMEM),
    )(protos[:_NUM_CLASSES])
    return out[0, 0]


# speedup vs baseline: 8.9418x; 1.2680x over previous
import functools

import jax
import jax.numpy as jnp
from jax import lax
from jax.experimental import pallas as pl
from jax.experimental.pallas import tpu as pltpu
from jax.experimental.pallas import tpu_sc as plsc

_NUM_CLASSES = 1000
_FEAT = 128
_BATCH = 4096
_M = 0.99
_TEMP = 0.1
_BASE_TEMP = 0.1

_NW = 32
_CPW = 32
_CPAD = _NW * _CPW
_NVEC = _FEAT // 16
_MIDX_SZ = _BATCH + 16


def _sc_ema_body(feat_hbm, lab_hbm, proto_hbm, out_hbm,
                 lab_v, midx_v, prot_v, rowb, feat_sh, sem):
    sid = lax.axis_index("s")
    wid = sid * 2 + lax.axis_index("c")
    lo = wid * _CPW
    hi = lo + _CPW

    pltpu.sync_copy(lab_hbm, lab_v.at[pl.ds(0, _BATCH)])
    pltpu.sync_copy(proto_hbm.at[pl.ds(lo, _CPW)], prot_v)

    @pl.when(sid == 0)
    def _():
        pltpu.sync_copy(feat_hbm, feat_sh)

    zero16 = jnp.zeros((16,), jnp.int32)

    pow2 = jnp.left_shift(jnp.int32(1), lax.iota(jnp.int32, 16))

    def sbody(i, cnt):
        base = i * 16
        lv = lab_v[pl.ds(pl.multiple_of(base, 16), 16)]
        m = (lv >= lo) & (lv < hi)
        w = jnp.where(m, pow2, zero16)
        pr = w + lax.rev(w, (0,))
        bits = pr[0]
        for k in range(1, 8):
            bits = bits + pr[k]

        def append(c):
            for k in range(16):
                midx_v[pl.ds(c, 16)] = jnp.full((16,), base + k, jnp.int32)
                c = c + ((bits >> k) & 1)
            return c

        return lax.cond(bits != 0, append, lambda c: c, cnt)

    cnt = lax.fori_loop(0, _BATCH // 16, sbody, 0, unroll=False)

    plsc.subcore_barrier()

    def batch(bi, c):
        b0 = pl.multiple_of(bi * 16, 16)
        iv = midx_v[pl.ds(b0, 16)]
        iv = jnp.minimum(jnp.maximum(iv, 0), _BATCH - 1)
        for k in range(16):
            pltpu.async_copy(feat_sh.at[pl.ds(iv[k], 1)],
                             rowb.at[pl.ds(k, 1)], sem)
        pltpu.make_async_copy(feat_sh.at[pl.ds(0, 16)], rowb, sem).wait()
        jhi = jnp.minimum(cnt - b0, 16)

        def ebody(j, cc):
            idx = midx_v[pl.ds(b0 + j, 16)][0]
            lloc = lab_v[pl.ds(idx, 16)][0] - lo
            ts = []
            ss = jnp.zeros((16,), jnp.float32)
            for k in range(_NVEC):
                p = prot_v[lloc, pl.ds(k * 16, 16)]
                f = rowb[j, pl.ds(k * 16, 16)]
                t = p * _M + f * (1.0 - _M)
                ts.append(t)
                ss = ss + t * t
            ss = ss + lax.rev(ss, (0,))
            s = ss[0]
            for k in range(1, 8):
                s = s + ss[k]
            s = jnp.maximum(s, 1e-30)
            ib = lax.bitcast_convert_type(s, jnp.int32)
            y = lax.bitcast_convert_type(
                jnp.int32(0x5F3759DF) - (ib >> 1), jnp.float32)
            for _ in range(3):
                y = y * (1.5 - 0.5 * s * y * y)
            nrm = s * y
            scale = jnp.where(nrm > 1e-12, y, 1e12)
            for k in range(_NVEC):
                prot_v[lloc, pl.ds(k * 16, 16)] = ts[k] * scale
            return cc

        lax.fori_loop(0, jhi, ebody, 0, unroll=False)
        return c

    nbatch = (cnt + 15) // 16
    lax.fori_loop(0, nbatch, batch, 0, unroll=False)

    pltpu.sync_copy(prot_v, out_hbm.at[pl.ds(lo, _CPW)])


_sc_ema = functools.partial(
    pl.kernel,
    out_type=jax.ShapeDtypeStruct((_CPAD, _FEAT), jnp.float32),
    mesh=plsc.VectorSubcoreMesh(core_axis_name="c", subcore_axis_name="s"),
    scratch_types=[
        pltpu.VMEM((_BATCH + 16,), jnp.int32),
        pltpu.VMEM((_MIDX_SZ,), jnp.int32),
        pltpu.VMEM((_CPW, _FEAT), jnp.float32),
        pltpu.VMEM((16, _FEAT), jnp.float32),
        pltpu.VMEM_SHARED((_BATCH, _FEAT), jnp.float32),
        pltpu.SemaphoreType.DMA,
    ],
)(_sc_ema_body)


def _loss_body(proto_ref, out_ref):
    p = proto_ref[...]
    logits = lax.dot_general(
        p, p, (((1,), (1,)), ((), ())), preferred_element_type=jnp.float32
    ) * (1.0 / _TEMP)
    e = jnp.exp(logits)
    ii = lax.broadcasted_iota(jnp.int32, (_NUM_CLASSES, _NUM_CLASSES), 0)
    jj = lax.broadcasted_iota(jnp.int32, (_NUM_CLASSES, _NUM_CLASSES), 1)
    offdiag = ii != jj
    rowsum = jnp.sum(jnp.where(offdiag, e, 0.0), axis=1)
    diag_e = jnp.max(jnp.where(offdiag, 0.0, e), axis=1)
    mpn = jnp.log(rowsum * (1.0 / (_NUM_CLASSES - 1)))
    valid = jnp.isfinite(diag_e)
    num = jnp.sum(jnp.where(valid, mpn, 0.0))
    den = jnp.maximum(jnp.sum(valid.astype(jnp.int32)), 1).astype(jnp.float32)
    out_ref[0, 0] = (_TEMP / _BASE_TEMP) * num / den


def kernel(features, labels, prototypes):
    labels = labels.astype(jnp.int32)
    protos_pad = jnp.concatenate(
        [prototypes, jnp.zeros((_CPAD - _NUM_CLASSES, _FEAT), jnp.float32)], 0
    )
    protos = _sc_ema(features, labels, protos_pad)
    out = pl.pallas_call(
        _loss_body,
        out_shape=jax.ShapeDtypeStruct((1, 1), jnp.float32),
        in_specs=[pl.BlockSpec(memory_space=pltpu.VMEM)],
        out_specs=pl.BlockSpec(memory_space=pltpu.SMEM),
    )(protos[:_NUM_CLASSES])
    return out[0, 0]
